# baseline (speedup 1.0000x reference)
import functools

import jax
import jax.numpy as jnp
from jax.experimental import pallas as pl
from jax.experimental.pallas import tpu as pltpu

B, L, H = 8, 2048, 1024
PC = 256


def _seg_sum_body(ids_ref, seq_ref, out_ref):
    tok_col = jax.lax.broadcasted_iota(jnp.int32, (L, PC), 0)
    ids = ids_ref[0, 0, :]
    acc = jnp.zeros((L, H), jnp.float32)
    for k in range(L // PC):
        ids_chunk = ids[k * PC:(k + 1) * PC]
        onehot = (tok_col == ids_chunk[None, :]).astype(jnp.bfloat16)
        chunk = seq_ref[0, k * PC:(k + 1) * PC, :].astype(jnp.bfloat16)
        acc += jnp.dot(onehot, chunk, preferred_element_type=jnp.float32)
    out_ref[0, :, :] = acc


@jax.jit
def kernel(sequence_output, wp_to_tok):
    ids3 = wp_to_tok.astype(jnp.int32).reshape(B, 1, L)
    out = pl.pallas_call(
        _seg_sum_body,
        grid=(B,),
        in_specs=[
            pl.BlockSpec((1, 1, L), lambda i: (i, 0, 0)),
            pl.BlockSpec((1, L, H), lambda i: (i, 0, 0)),
        ],
        out_specs=pl.BlockSpec((1, L, H), lambda i: (i, 0, 0)),
        out_shape=jax.ShapeDtypeStruct((B, L, H), jnp.float32),
    )(ids3, sequence_output)
    return out


# speedup vs baseline: 1.0059x; 1.0059x over previous
import functools

import jax
import jax.numpy as jnp
from jax.experimental import pallas as pl
from jax.experimental.pallas import tpu as pltpu

B, L, H = 8, 2048, 1024
PC = 512


def _seg_sum_body(ids_ref, seq_ref, out_ref):
    tok_col = jax.lax.broadcasted_iota(jnp.int32, (L, PC), 0)
    ids = ids_ref[0, 0, :]
    acc = jnp.zeros((L, H), jnp.float32)
    for k in range(L // PC):
        ids_chunk = ids[k * PC:(k + 1) * PC]
        onehot = (tok_col == ids_chunk[None, :]).astype(jnp.float32)
        chunk = seq_ref[0, k * PC:(k + 1) * PC, :]
        acc += jnp.dot(onehot, chunk, preferred_element_type=jnp.float32)
    out_ref[0, :, :] = acc


@jax.jit
def kernel(sequence_output, wp_to_tok):
    ids3 = wp_to_tok.astype(jnp.int32).reshape(B, 1, L)
    out = pl.pallas_call(
        _seg_sum_body,
        grid=(B,),
        in_specs=[
            pl.BlockSpec((1, 1, L), lambda i: (i, 0, 0)),
            pl.BlockSpec((1, L, H), lambda i: (i, 0, 0)),
        ],
        out_specs=pl.BlockSpec((1, L, H), lambda i: (i, 0, 0)),
        out_shape=jax.ShapeDtypeStruct((B, L, H), jnp.float32),
    )(ids3, sequence_output)
    return out


# speedup vs baseline: 1.0064x; 1.0005x over previous
import functools

import jax
import jax.numpy as jnp
from jax.experimental import pallas as pl
from jax.experimental.pallas import tpu as pltpu

B, L, H = 8, 2048, 1024
PC = 512


def _seg_sum_body(ids_ref, seq_ref, out_ref):
    tok_col = jax.lax.broadcasted_iota(jnp.int32, (L, L), 0)
    ids = ids_ref[0, 0, :]
    onehot = (tok_col == ids[None, :]).astype(jnp.float32)
    out_ref[0, :, :] = jnp.dot(onehot, seq_ref[0, :, :],
                               preferred_element_type=jnp.float32)


@jax.jit
def kernel(sequence_output, wp_to_tok):
    ids3 = wp_to_tok.astype(jnp.int32).reshape(B, 1, L)
    out = pl.pallas_call(
        _seg_sum_body,
        grid=(B,),
        in_specs=[
            pl.BlockSpec((1, 1, L), lambda i: (i, 0, 0)),
            pl.BlockSpec((1, L, H), lambda i: (i, 0, 0)),
        ],
        out_specs=pl.BlockSpec((1, L, H), lambda i: (i, 0, 0)),
        out_shape=jax.ShapeDtypeStruct((B, L, H), jnp.float32),
    )(ids3, sequence_output)
    return out


# speedup vs baseline: 1.0161x; 1.0097x over previous
import functools

import jax
import jax.numpy as jnp
from jax.experimental import pallas as pl
from jax.experimental.pallas import tpu as pltpu

B, L, H = 8, 2048, 1024
PC = 512


def _seg_sum_body(ids_ref, seq_ref, out_ref):
    tok_col = jax.lax.broadcasted_iota(jnp.int32, (L, L), 0)
    ids = ids_ref[0, 0, :]
    onehot = (tok_col == ids[None, :]).astype(jnp.float32)
    out_ref[0, :, :] = jnp.dot(onehot, seq_ref[0, :, :],
                               preferred_element_type=jnp.float32)


@jax.jit
def kernel(sequence_output, wp_to_tok):
    ids3 = wp_to_tok.astype(jnp.int32).reshape(B, 1, L)
    hc = H // 2
    out = pl.pallas_call(
        _seg_sum_body,
        grid=(B, 2),
        in_specs=[
            pl.BlockSpec((1, 1, L), lambda i, j: (i, 0, 0)),
            pl.BlockSpec((1, L, hc), lambda i, j: (i, 0, j)),
        ],
        out_specs=pl.BlockSpec((1, L, hc), lambda i, j: (i, 0, j)),
        out_shape=jax.ShapeDtypeStruct((B, L, H), jnp.float32),
    )(ids3, sequence_output)
    return out
